# baseline (speedup 1.0000x reference)
import functools
import math

import jax
import jax.numpy as jnp
from jax import lax
from jax.experimental import pallas as pl
from jax.experimental.pallas import tpu as pltpu
from jax.experimental.pallas import tpu_sc as plsc

VOCAB = 1000000
EMB = 64
SCALE = math.sqrt(EMB)

_info = plsc.get_sparse_core_info()
NC, NS, L = _info.num_cores, _info.num_subcores, _info.num_lanes
NW = NC * NS

CHUNK = 512
IDX_PER_DMA = 128
N_DMA = CHUNK // IDX_PER_DMA


def _emb_kernel(n_rows, tokens_hbm, table_hbm, out_hbm, idx_v, rows_v, sem):
    wid = lax.axis_index("s") * NC + lax.axis_index("c")
    per_w = n_rows // NW
    n_chunks = per_w // CHUNK
    base = wid * per_w

    def chunk_body(g, _):
        row0 = base + g * CHUNK
        pltpu.sync_copy(tokens_hbm.at[pl.ds(row0, CHUNK)], idx_v)
        copies = []
        for j in range(N_DMA):
            copies.append(pltpu.async_copy(
                table_hbm.at[idx_v.at[pl.ds(j * IDX_PER_DMA, IDX_PER_DMA)]],
                rows_v.at[pl.ds(j * IDX_PER_DMA, IDX_PER_DMA)],
                sem))
        for c in copies:
            c.wait()

        @plsc.parallel_loop(0, CHUNK, 1, unroll=8)
        def _scale(r):
            for j in range(EMB // L):
                sl = pl.ds(j * L, L)
                rows_v[r, sl] = rows_v[r, sl] * SCALE

        pltpu.sync_copy(rows_v, out_hbm.at[pl.ds(row0, CHUNK)])
        return 0

    lax.fori_loop(0, n_chunks, chunk_body, 0)


def kernel(tokens, table):
    orig_shape = tokens.shape
    tokens_flat = tokens.reshape(-1).astype(jnp.int32)
    n_rows = tokens_flat.shape[0]
    assert n_rows % (NW * CHUNK) == 0

    mesh = plsc.VectorSubcoreMesh(core_axis_name="c", subcore_axis_name="s")
    f = pl.kernel(
        functools.partial(_emb_kernel, n_rows),
        out_type=jax.ShapeDtypeStruct((n_rows, EMB), jnp.float32),
        mesh=mesh,
        scratch_types=[
            pltpu.VMEM((CHUNK,), jnp.int32),
            pltpu.VMEM((CHUNK, EMB), jnp.float32),
            pltpu.SemaphoreType.DMA,
        ],
        compiler_params=pltpu.CompilerParams(use_tc_tiling_on_sc=False),
    )
    out = f(tokens_flat, table)
    return out.reshape(*orig_shape, EMB)


# speedup vs baseline: 1.0875x; 1.0875x over previous
import functools
import math

import jax
import jax.numpy as jnp
from jax import lax
from jax.experimental import pallas as pl
from jax.experimental.pallas import tpu as pltpu
from jax.experimental.pallas import tpu_sc as plsc

VOCAB = 1000000
EMB = 64
SCALE = math.sqrt(EMB)

_info = plsc.get_sparse_core_info()
NC, NS, L = _info.num_cores, _info.num_subcores, _info.num_lanes
NW = NC * NS

CHUNK = 256
IDX_PER_DMA = 128
N_DMA = CHUNK // IDX_PER_DMA
DEPTH = 4
DIST = 2


def _emb_kernel(n_rows, tokens_hbm, table_hbm, out_hbm,
                idx_v, rows_v, gsem, ssem):
    wid = lax.axis_index("s") * NC + lax.axis_index("c")
    per_w = n_rows // NW
    n_chunks = per_w // CHUNK
    base = wid * per_w

    pltpu.sync_copy(tokens_hbm.at[pl.ds(base, per_w)], idx_v)

    def fire_gathers(c, b):
        for j in range(N_DMA):
            pltpu.async_copy(
                table_hbm.at[idx_v.at[pl.ds(c * CHUNK + j * IDX_PER_DMA,
                                            IDX_PER_DMA)]],
                rows_v.at[b, pl.ds(j * IDX_PER_DMA, IDX_PER_DMA)],
                gsem.at[b])

    def wait_gathers(b):
        pltpu.make_async_copy(
            table_hbm.at[pl.ds(0, CHUNK)], rows_v.at[b], gsem.at[b]).wait()

    def fire_store(c, b):
        pltpu.async_copy(
            rows_v.at[b], out_hbm.at[pl.ds(base + c * CHUNK, CHUNK)],
            ssem.at[b])

    def wait_store(b):
        pltpu.make_async_copy(
            rows_v.at[b], out_hbm.at[pl.ds(0, CHUNK)], ssem.at[b]).wait()

    def scale(b):
        @plsc.parallel_loop(0, CHUNK, 1, unroll=8)
        def _scale(r):
            for j in range(EMB // L):
                sl = pl.ds(j * L, L)
                rows_v[b, r, sl] = rows_v[b, r, sl] * SCALE

    def chunk_step(c, b, wait_prev_store, fire_next):
        wait_gathers(b)
        scale(b)
        fire_store(c, b)
        if fire_next:
            b2 = (b + DIST) % DEPTH
            if wait_prev_store:
                wait_store(b2)
            fire_gathers(c + DIST, b2)

    for b in range(DIST):
        fire_gathers(b, b)
    for c in range(DIST):
        chunk_step(c, c % DEPTH, False, True)
    for c in range(DIST, DEPTH):
        chunk_step(c, c % DEPTH, True, True)

    @pl.loop(DEPTH, n_chunks - DEPTH, step=DEPTH)
    def body(g):
        for b in range(DEPTH):
            chunk_step(g + b, b, True, True)

    for k in range(DEPTH, DIST, -1):
        chunk_step(n_chunks - k, (n_chunks - k) % DEPTH, True, True)
    for k in range(DIST, 0, -1):
        chunk_step(n_chunks - k, (n_chunks - k) % DEPTH, False, False)

    for b in range(DEPTH):
        wait_store(b)


def kernel(tokens, table):
    orig_shape = tokens.shape
    tokens_flat = tokens.reshape(-1).astype(jnp.int32)
    n_rows = tokens_flat.shape[0]
    per_w = n_rows // NW
    assert n_rows % (NW * CHUNK) == 0
    assert (per_w // CHUNK) % DEPTH == 0

    mesh = plsc.VectorSubcoreMesh(core_axis_name="c", subcore_axis_name="s")
    f = pl.kernel(
        functools.partial(_emb_kernel, n_rows),
        out_type=jax.ShapeDtypeStruct((n_rows, EMB), jnp.float32),
        mesh=mesh,
        scratch_types=[
            pltpu.VMEM((per_w,), jnp.int32),
            pltpu.VMEM((DEPTH, CHUNK, EMB), jnp.float32),
            pltpu.SemaphoreType.DMA((DEPTH,)),
            pltpu.SemaphoreType.DMA((DEPTH,)),
        ],
        compiler_params=pltpu.CompilerParams(use_tc_tiling_on_sc=False),
    )
    out = f(tokens_flat, table)
    return out.reshape(*orig_shape, EMB)


# speedup vs baseline: 2.5537x; 2.3483x over previous
import functools
import math

import jax
import jax.numpy as jnp
from jax import lax
from jax.experimental import pallas as pl
from jax.experimental.pallas import tpu as pltpu
from jax.experimental.pallas import tpu_sc as plsc

EMB = 64
SCALE = math.sqrt(EMB)

_info = plsc.get_sparse_core_info()
NC, NS, L = _info.num_cores, _info.num_subcores, _info.num_lanes
NW = NC * NS

BC = 8192
TB = 128
DG = 4
DT = 2
DIST = 2


def _transpose_body(tab_t_ref, out_ref):
    out_ref[:, :EMB] = tab_t_ref[...].T * SCALE


SKEW = EMB + 1


def _gather_kernel(n_cols, tok_t_hbm, tab_hbm, out_hbm,
                   idx_v, gbuf, tbuf, sbuf, gsem, tsem):
    wid = lax.axis_index("s") * NC + lax.axis_index("c")
    col0 = wid * TB

    pltpu.sync_copy(tok_t_hbm.at[:, pl.ds(col0, TB)], idx_v)

    def fire_gather(c, gb):
        pltpu.async_copy(tab_hbm.at[idx_v.at[c]], gbuf.at[gb], gsem.at[gb])

    def wait_gather(gb):
        pltpu.make_async_copy(
            tab_hbm.at[pl.ds(0, TB)], gbuf.at[gb], gsem.at[gb]).wait()

    def fire_tstore(c, tb):
        pltpu.async_copy(
            tbuf.at[tb], out_hbm.at[c, :, pl.ds(col0, TB)], tsem.at[tb])

    def wait_tstore(tb):
        pltpu.make_async_copy(
            tbuf.at[tb], out_hbm.at[0, :, pl.ds(0, TB)], tsem.at[tb]).wait()

    ivec = jnp.arange(16, dtype=jnp.int32) * SKEW

    def extract(gb, tb):
        @plsc.parallel_loop(0, TB, 1, unroll=4)
        def _t(k):
            for g in range(EMB // 16):
                sbuf[pl.ds(k * SKEW + g * 16, 16)] = \
                    gbuf[gb, k, pl.ds(g * 16, 16)]

        @plsc.parallel_loop(0, EMB, 1, unroll=4)
        def _t2(f):
            for kb in range(TB // 16):
                vals = plsc.load_gather(sbuf, [ivec + (kb * 16 * SKEW + f)])
                tbuf[tb, f, pl.ds(kb * 16, 16)] = vals

    def step(c, gb, tb, wait_ts, fire_g):
        wait_gather(gb)
        if wait_ts:
            wait_tstore(tb)
        extract(gb, tb)
        fire_tstore(c, tb)
        if fire_g:
            fire_gather(c + DIST, (c + DIST) % DG)

    for b in range(DIST):
        fire_gather(b, b)
    for c in range(DIST):
        step(c, c % DG, c % DT, False, True)
    for c in range(DIST, DG):
        step(c, c % DG, c % DT, True, True)

    @pl.loop(DG, n_cols - DG, step=DG)
    def body(g):
        for b in range(DG):
            step(g + b, b, (g + b) % DT, True, True)

    for k in range(DG, DIST, -1):
        c = n_cols - k
        step(c, c % DG, c % DT, True, True)
    for k in range(DIST, 0, -1):
        c = n_cols - k
        step(c, c % DG, c % DT, True, False)

    for tb in range(DT):
        wait_tstore(tb)


def kernel(tokens, table):
    B, T = tokens.shape
    V, E = table.shape
    assert E == EMB and B // NW == TB and T % DG == 0 and T >= 2 * DG

    tok_t = tokens.T
    tab_t = table.T

    tab_rm = pl.pallas_call(
        _transpose_body,
        grid=(pl.cdiv(V, BC),),
        in_specs=[pl.BlockSpec((E, BC), lambda i: (0, i))],
        out_specs=pl.BlockSpec((BC, 2 * E), lambda i: (i, 0)),
        out_shape=jax.ShapeDtypeStruct((V, 2 * E), jnp.float32),
    )(tab_t)

    mesh = plsc.VectorSubcoreMesh(core_axis_name="c", subcore_axis_name="s")
    f = pl.kernel(
        functools.partial(_gather_kernel, T),
        out_type=jax.ShapeDtypeStruct((T, EMB, B), jnp.float32),
        mesh=mesh,
        scratch_types=[
            pltpu.VMEM((T, TB), jnp.int32),
            pltpu.VMEM((DG, TB, 2 * EMB), jnp.float32),
            pltpu.VMEM((DT, EMB, TB), jnp.float32),
            pltpu.VMEM((TB * SKEW,), jnp.float32),
            pltpu.SemaphoreType.DMA((DG,)),
            pltpu.SemaphoreType.DMA((DT,)),
        ],
        compiler_params=pltpu.CompilerParams(
            use_tc_tiling_on_sc=True, needs_layout_passes=False),
    )
    out_t = f(tok_t, tab_rm)
    return jnp.transpose(out_t, (2, 0, 1))
